# Initial kernel scaffold; baseline (speedup 1.0000x reference)
import jax
import jax.numpy as jnp
from jax.experimental import pallas as pl


def kernel(x, edge_index, Ws_shared, bs_shared, Ws_mu, bs_mu, Ws_logvar, bs_logvar):
    raise NotImplementedError("write your pallas kernel here")


# speedup vs baseline: 2.9345x; 2.9345x over previous
import functools

import jax
import jax.numpy as jnp
from jax import lax
from jax.experimental import pallas as pl
from jax.experimental.pallas import tpu as pltpu
from jax.experimental.pallas import tpu_sc as plsc

_N = 10000
_E = 320000
_D = 128

_NC = 2
_NS = 16
_NW = _NC * _NS

_CH = 128
_NCH = (_E // _NW + _CH - 1) // _CH
_NCH = ((_NCH + 3) // 4) * 4
_HNCH = _NCH // 2
_EPW = _NCH * _CH
_EPAD = _EPW * _NW
_RPT = 640
_NPAD = _RPT * _NS
_ZB = 32

_HIGHEST = lax.Precision.HIGHEST


def _segment_sum_sc(h, src2d, dst2d):
    mesh = plsc.VectorSubcoreMesh(core_axis_name="c", subcore_axis_name="s")

    @functools.partial(
        pl.kernel,
        out_type=jax.ShapeDtypeStruct((_NC, _NPAD, _D), jnp.float32),
        mesh=mesh,
        scratch_types=[
            pltpu.VMEM((_NCH, _CH), jnp.int32),
            pltpu.VMEM((_HNCH, _CH), jnp.int32),
            pltpu.VMEM((_CH, _D), jnp.float32),
            pltpu.VMEM((_CH, _D), jnp.float32),
            pltpu.VMEM_SHARED((_NPAD, _D), jnp.float32),
            pltpu.SemaphoreType.DMA,
            pltpu.SemaphoreType.DMA,
        ],
    )
    def ssum(h_hbm, src_hbm, dst_hbm, out_hbm,
             src_v, dst_v, buf0, buf1, acc, sem0, sem1):
        cid = lax.axis_index("c")
        sid = lax.axis_index("s")
        wid = sid * _NC + cid

        pltpu.sync_copy(src_hbm.at[wid], src_v)

        z16 = jnp.zeros((16,), jnp.float32)
        for r in range(_ZB):
            for c in range(_D // 16):
                buf0[r, pl.ds(c * 16, 16)] = z16

        def zero_body(i, carry):
            pltpu.sync_copy(buf0.at[pl.ds(0, _ZB)],
                            acc.at[pl.ds(sid * _RPT + i * _ZB, _ZB)])
            return carry
        lax.fori_loop(0, _RPT // _ZB, zero_body, 0)
        plsc.subcore_barrier()

        bufs = (buf0, buf1)
        sems = (sem0, sem1)

        pltpu.async_copy(h_hbm.at[src_v.at[0]], buf0, sem0)
        pltpu.async_copy(h_hbm.at[src_v.at[1]], buf1, sem1)

        def pair(j, lj, restart):
            for b in range(2):
                pltpu.make_async_copy(h_hbm.at[src_v.at[j + b]], bufs[b],
                                      sems[b]).wait()
                pltpu.sync_copy(bufs[b], acc.at[dst_v.at[lj + b]], add=True)
                if restart:
                    pltpu.async_copy(h_hbm.at[src_v.at[j + b + 2]], bufs[b],
                                     sems[b])

        for half in range(2):
            pltpu.sync_copy(dst_hbm.at[wid, pl.ds(half * _HNCH, _HNCH)],
                            dst_v)
            npairs = _HNCH // 2 - (1 if half == 1 else 0)
            lax.fori_loop(
                0, npairs,
                lambda i, c, _h=half: (pair(_h * _HNCH + i * 2, i * 2, True),
                                       c)[1],
                0)

        pair(_NCH - 2, _HNCH - 2, False)

        plsc.subcore_barrier()
        pltpu.sync_copy(acc.at[pl.ds(sid * _RPT, _RPT)],
                        out_hbm.at[cid, pl.ds(sid * _RPT, _RPT)])

    return ssum(h, src2d, dst2d)


def _leaky(v):
    return jnp.where(v > 0, v, 0.1 * v)


_BN = 1000


def _mlp_shared_tc(x, parts, wt, b):
    def body(x_ref, p_ref, w_ref, b_ref, o_ref):
        z = x_ref[...] + p_ref[0] + p_ref[1]
        for i in range(4):
            z = jnp.dot(z, w_ref[i], preferred_element_type=jnp.float32,
                        precision=_HIGHEST) + b_ref[i]
            if i < 3:
                z = _leaky(z)
        o_ref[...] = jnp.maximum(z, 0.0)

    return pl.pallas_call(
        body,
        grid=(_N // _BN,),
        in_specs=[
            pl.BlockSpec((_BN, _D), lambda i: (i, 0)),
            pl.BlockSpec((_NC, _BN, _D), lambda i: (0, i, 0)),
            pl.BlockSpec((4, _D, _D), lambda i: (0, 0, 0)),
            pl.BlockSpec((4, _D), lambda i: (0, 0)),
        ],
        out_specs=pl.BlockSpec((_BN, _D), lambda i: (i, 0)),
        out_shape=jax.ShapeDtypeStruct((_N, _D), jnp.float32),
    )(x, parts, wt, b)


def _mlp_heads_tc(h, parts, wmu_t, bmu, wlv_t, blv):
    def body(h_ref, p_ref, wm_ref, bm_ref, wl_ref, bl_ref, mu_ref, lv_ref):
        z = h_ref[...] + p_ref[0] + p_ref[1]
        t = jnp.maximum(jnp.dot(z, wm_ref[0], preferred_element_type=jnp.float32,
                                precision=_HIGHEST) + bm_ref[0], 0.0)
        mu_ref[...] = jnp.dot(t, wm_ref[1], preferred_element_type=jnp.float32,
                              precision=_HIGHEST) + bm_ref[1]
        t = jnp.maximum(jnp.dot(z, wl_ref[0], preferred_element_type=jnp.float32,
                                precision=_HIGHEST) + bl_ref[0], 0.0)
        lv_ref[...] = jnp.dot(t, wl_ref[1], preferred_element_type=jnp.float32,
                              precision=_HIGHEST) + bl_ref[1]

    return pl.pallas_call(
        body,
        grid=(_N // _BN,),
        in_specs=[
            pl.BlockSpec((_BN, _D), lambda i: (i, 0)),
            pl.BlockSpec((_NC, _BN, _D), lambda i: (0, i, 0)),
            pl.BlockSpec((2, _D, _D), lambda i: (0, 0, 0)),
            pl.BlockSpec((2, _D), lambda i: (0, 0)),
            pl.BlockSpec((2, _D, _D), lambda i: (0, 0, 0)),
            pl.BlockSpec((2, _D), lambda i: (0, 0)),
        ],
        out_specs=[
            pl.BlockSpec((_BN, _D), lambda i: (i, 0)),
            pl.BlockSpec((_BN, _D), lambda i: (i, 0)),
        ],
        out_shape=[
            jax.ShapeDtypeStruct((_N, _D), jnp.float32),
            jax.ShapeDtypeStruct((_N, _D), jnp.float32),
        ],
    )(h, parts, wmu_t, bmu, wlv_t, blv)


def kernel(x, edge_index, Ws_shared, bs_shared, Ws_mu, bs_mu, Ws_logvar,
           bs_logvar):
    pad = _EPAD - _E
    src = jnp.concatenate([edge_index[0], jnp.zeros((pad,), jnp.int32)])
    dst = jnp.concatenate([edge_index[1], jnp.full((pad,), _N, jnp.int32)])
    src2d = src.reshape(_NW, _NCH, _CH)
    dst2d = dst.reshape(_NW, _NCH, _CH)

    parts1 = _segment_sum_sc(x, src2d, dst2d)
    h = _mlp_shared_tc(x, parts1, Ws_shared.transpose(0, 2, 1), bs_shared)
    parts2 = _segment_sum_sc(h, src2d, dst2d)
    mu, logvar = _mlp_heads_tc(h, parts2, Ws_mu.transpose(0, 2, 1), bs_mu,
                               Ws_logvar.transpose(0, 2, 1), bs_logvar)
    return (mu, logvar)
